# baseline (speedup 1.0000x reference)
import functools

import jax
import jax.numpy as jnp
from jax import lax
from jax.experimental import pallas as pl
from jax.experimental.pallas import tpu as pltpu

IN_C = 384
OUT_C = 192
HW = 224 * 224


def _topk_body(p_row_ref, p_col_ref, vals_ref, idx_ref):
    p_row = p_row_ref[...]
    p_col = p_col_ref[...]
    gt = (p_row > p_col).astype(jnp.int32)
    jj = lax.broadcasted_iota(jnp.int32, (IN_C, IN_C), 1)
    ii = lax.broadcasted_iota(jnp.int32, (IN_C, IN_C), 0)
    tie = ((p_row == p_col) & (jj < ii)).astype(jnp.int32)
    rank = jnp.sum(gt + tie, axis=1, keepdims=True)
    rr = lax.broadcasted_iota(jnp.int32, (IN_C, OUT_C), 1)
    m = rank == rr
    vals_ref[...] = jnp.sum(
        jnp.where(m, p_col, jnp.float32(0)), axis=0, keepdims=True)
    ii_c = lax.broadcasted_iota(jnp.int32, (IN_C, OUT_C), 0)
    idx_ref[...] = jnp.sum(
        jnp.where(m, ii_c, 0), axis=0, keepdims=True)


def _topk(params):
    p_row = params.reshape(1, IN_C)
    p_col = params.reshape(IN_C, 1)
    vals, idx = pl.pallas_call(
        _topk_body,
        out_shape=(
            jax.ShapeDtypeStruct((1, OUT_C), jnp.float32),
            jax.ShapeDtypeStruct((1, OUT_C), jnp.int32),
        ),
    )(p_row, p_col)
    return vals.reshape(OUT_C), idx.reshape(OUT_C)


def _gather_body(idx_ref, w_ref, x_ref, o_ref):
    i = pl.program_id(0)
    w = w_ref[i % OUT_C]
    o_ref[...] = w * x_ref[...]


def _gather(x3d, idx, w):
    grid_spec = pltpu.PrefetchScalarGridSpec(
        num_scalar_prefetch=2,
        grid=(2 * OUT_C,),
        in_specs=[
            pl.BlockSpec(
                (1, HW // 128, 128),
                lambda i, idx_ref, w_ref: (
                    (i // OUT_C) * IN_C + idx_ref[i % OUT_C], 0, 0),
            ),
        ],
        out_specs=pl.BlockSpec(
            (1, HW // 128, 128), lambda i, idx_ref, w_ref: (i, 0, 0)),
    )
    return pl.pallas_call(
        _gather_body,
        grid_spec=grid_spec,
        out_shape=jax.ShapeDtypeStruct((2 * OUT_C, HW // 128, 128), jnp.float32),
    )(idx, w, x3d)


@jax.jit
def kernel(x, params):
    b = x.shape[0]
    w, idx = _topk(params)
    x3d = x.reshape(b * IN_C, HW // 128, 128)
    out = _gather(x3d, idx, w)
    return out.reshape(b, OUT_C, 224, 224)


# speedup vs baseline: 1.7004x; 1.7004x over previous
import functools

import jax
import jax.numpy as jnp
from jax import lax
from jax.experimental import pallas as pl
from jax.experimental.pallas import tpu as pltpu

IN_C = 384
OUT_C = 192
HW = 224 * 224


def _topk_body(p_row_ref, p_col_ref, vals_ref, idx_ref):
    p_row = p_row_ref[...]
    p_col = p_col_ref[...]
    gt = (p_row > p_col).astype(jnp.int32)
    jj = lax.broadcasted_iota(jnp.int32, (IN_C, IN_C), 1)
    ii = lax.broadcasted_iota(jnp.int32, (IN_C, IN_C), 0)
    tie = ((p_row == p_col) & (jj < ii)).astype(jnp.int32)
    rank = jnp.sum(gt + tie, axis=1, keepdims=True)
    rr = lax.broadcasted_iota(jnp.int32, (IN_C, OUT_C), 1)
    m = rank == rr
    vals_ref[...] = jnp.sum(
        jnp.where(m, p_col, jnp.float32(0)), axis=0, keepdims=True)
    ii_c = lax.broadcasted_iota(jnp.int32, (IN_C, OUT_C), 0)
    idx_ref[...] = jnp.sum(
        jnp.where(m, ii_c, 0), axis=0, keepdims=True)


def _topk(params):
    p_row = params.reshape(1, IN_C)
    p_col = params.reshape(IN_C, 1)
    vals, idx = pl.pallas_call(
        _topk_body,
        out_shape=(
            jax.ShapeDtypeStruct((1, OUT_C), jnp.float32),
            jax.ShapeDtypeStruct((1, OUT_C), jnp.int32),
        ),
    )(p_row, p_col)
    return vals.reshape(OUT_C), idx.reshape(OUT_C)


def _gather_body(idx_ref, w_ref, x_ref, o_ref):
    i = pl.program_id(0)
    w = w_ref[i % OUT_C]
    o_ref[...] = w * x_ref[...]


def _gather(x, idx, w):
    grid_spec = pltpu.PrefetchScalarGridSpec(
        num_scalar_prefetch=2,
        grid=(2 * OUT_C,),
        in_specs=[
            pl.BlockSpec(
                (1, 1, 224, 224),
                lambda i, idx_ref, w_ref: (
                    i // OUT_C, idx_ref[i % OUT_C], 0, 0),
            ),
        ],
        out_specs=pl.BlockSpec(
            (1, 1, 224, 224), lambda i, idx_ref, w_ref: (i // OUT_C, i % OUT_C, 0, 0)),
    )
    return pl.pallas_call(
        _gather_body,
        grid_spec=grid_spec,
        out_shape=jax.ShapeDtypeStruct((2, OUT_C, 224, 224), jnp.float32),
    )(idx, w, x)


@jax.jit
def kernel(x, params):
    w, idx = _topk(params)
    return _gather(x, idx, w)
